# baseline (speedup 1.0000x reference)
import functools

import jax
import jax.numpy as jnp
from jax import lax
from jax.experimental import pallas as pl
from jax.experimental.pallas import tpu as pltpu
from jax.experimental.pallas import tpu_sc as plsc

N = 10000
NP = 10240
E = 320000
IN_C = 128
HID_C = 256
OUT_C = 128

CH = 128
NC = 2
NS = 16
NW = NC * NS
NCHUNK = 80
HC = 40
EPW = NCHUNK * CH
EPAD = EPW * NW
RPT = NP // NS

_sc_mesh = plsc.VectorSubcoreMesh(core_axis_name="c", subcore_axis_name="s")


EDGW = E // NW


def _deg_body(ei_hbm, out_hbm, didx_all, deg_local, colbuf, shared):
    cid = lax.axis_index("c")
    sid = lax.axis_index("s")
    wid = sid * NC + cid
    zeros = jnp.zeros((16,), jnp.float32)

    def zbody(i, c):
        deg_local[pl.ds(i * 16, 16)] = zeros
        return c

    lax.fori_loop(0, NP // 16, zbody, 0)
    pltpu.sync_copy(ei_hbm.at[pl.ds(E + wid * EDGW, EDGW)], didx_all)
    ones = jnp.ones((16,), jnp.float32)

    def body(i, c):
        idx = didx_all[pl.ds(i * 16, 16)]
        plsc.addupdate_scatter(deg_local, [idx], ones)
        return c

    lax.fori_loop(0, EDGW // 16, body, 0)
    pltpu.sync_copy(deg_local, shared.at[sid])
    plsc.subcore_barrier()
    for t in range(NS):
        pltpu.sync_copy(shared.at[t, pl.ds(sid * RPT, RPT)], colbuf.at[t])

    def red(j, c):
        s = colbuf[0, pl.ds(j * 16, 16)]
        for t in range(1, NS):
            s = s + colbuf[t, pl.ds(j * 16, 16)]
        deg_local[pl.ds(j * 16, 16)] = s
        return c

    lax.fori_loop(0, RPT // 16, red, 0)
    pltpu.sync_copy(deg_local.at[pl.ds(0, RPT)],
                    out_hbm.at[pl.ds(cid * NP + sid * RPT, RPT)])


_deg_call = pl.kernel(
    _deg_body,
    out_type=jax.ShapeDtypeStruct((2 * NP,), jnp.float32),
    mesh=_sc_mesh,
    compiler_params=pltpu.CompilerParams(needs_layout_passes=False),
    scratch_types=[
        pltpu.VMEM((EDGW,), jnp.int32),
        pltpu.VMEM((NP,), jnp.float32),
        pltpu.VMEM((NS, RPT), jnp.float32),
        pltpu.VMEM_SHARED((NS, NP), jnp.float32),
    ],
)


def _agg_body(src_hbm, dst_hbm, g_hbm, zrows_hbm, out_hbm,
              sidx, didx, rows0, rows1, acc, sem0, sem1):
    cid = lax.axis_index("c")
    sid = lax.axis_index("s")
    wid = sid * NC + cid
    pltpu.sync_copy(zrows_hbm, acc.at[pl.ds(sid * RPT, RPT)])
    plsc.subcore_barrier()
    hpair = HC // 2
    for sb in range(NCHUNK // HC):
        base_c = wid * NCHUNK + sb * HC
        pltpu.sync_copy(src_hbm.at[pl.ds(base_c, HC)], sidx)
        pltpu.sync_copy(dst_hbm.at[pl.ds(base_c, HC)], didx)
        pltpu.async_copy(g_hbm.at[sidx.at[0]], rows0, sem0)

        def body(j, carry):
            c = 4 * j
            pltpu.async_copy(g_hbm.at[sidx.at[c + 1]], rows1, sem1)
            pltpu.make_async_copy(g_hbm.at[sidx.at[c]], rows0, sem0).wait()
            pltpu.sync_copy(rows0, acc.at[didx.at[c]], add=True)
            pltpu.async_copy(g_hbm.at[sidx.at[c + 2]], rows0, sem0)
            pltpu.make_async_copy(g_hbm.at[sidx.at[c + 1]], rows1, sem1).wait()
            pltpu.sync_copy(rows1, acc.at[didx.at[c + 1]], add=True)
            pltpu.async_copy(g_hbm.at[sidx.at[c + 3]], rows1, sem1)
            pltpu.make_async_copy(g_hbm.at[sidx.at[c + 2]], rows0, sem0).wait()
            pltpu.sync_copy(rows0, acc.at[didx.at[c + 2]], add=True)

            @pl.when(j < hpair // 2 - 1)
            def _():
                pltpu.async_copy(g_hbm.at[sidx.at[c + 4]], rows0, sem0)

            pltpu.make_async_copy(g_hbm.at[sidx.at[c + 3]], rows1, sem1).wait()
            pltpu.sync_copy(rows1, acc.at[didx.at[c + 3]], add=True)
            return carry

        lax.fori_loop(0, hpair // 2, body, 0)
    plsc.subcore_barrier()
    pltpu.sync_copy(acc.at[pl.ds(sid * RPT, RPT)],
                    out_hbm.at[pl.ds(cid * NP + sid * RPT, RPT)])


_agg_call = pl.kernel(
    _agg_body,
    out_type=jax.ShapeDtypeStruct((2 * NP, IN_C), jnp.float32),
    mesh=_sc_mesh,
    scratch_types=[
        pltpu.VMEM((HC, CH), jnp.int32),
        pltpu.VMEM((HC, CH), jnp.int32),
        pltpu.VMEM((CH, IN_C), jnp.float32),
        pltpu.VMEM((CH, IN_C), jnp.float32),
        pltpu.VMEM_SHARED((NP, IN_C), jnp.float32),
        pltpu.SemaphoreType.DMA,
        pltpu.SemaphoreType.DMA,
    ],
)


_RB = 2048


def _prep_body(degp_ref, x_ref, dinv_ref, g0_ref):
    nrow = _RB // 128
    deg = degp_ref[0] + degp_ref[1] + 1.0
    dinv = lax.rsqrt(jnp.maximum(deg, 1.0))
    drep = jnp.broadcast_to(dinv[:, None, :], (nrow, 128, 128)).reshape(_RB, 128)
    lane = lax.broadcasted_iota(jnp.int32, (_RB, 128), 1)
    row = lax.broadcasted_iota(jnp.int32, (_RB, 128), 0)
    sel = jnp.where(lane == row % 128, drep, 0.0)
    dinv_col = jnp.sum(sel, axis=1, keepdims=True)
    dinv_ref[...] = dinv_col
    g0_ref[...] = x_ref[...] * jnp.broadcast_to(dinv_col, (_RB, IN_C))


def _prep_call(degp, x_p):
    grid = NP // _RB
    return pl.pallas_call(
        _prep_body,
        grid=(grid,),
        in_specs=[
            pl.BlockSpec((2, _RB // 128, 128), lambda i: (0, i, 0)),
            pl.BlockSpec((_RB, IN_C), lambda i: (i, 0)),
        ],
        out_specs=[
            pl.BlockSpec((_RB, 1), lambda i: (i, 0)),
            pl.BlockSpec((_RB, IN_C), lambda i: (i, 0)),
        ],
        out_shape=[
            jax.ShapeDtypeStruct((NP, 1), jnp.float32),
            jax.ShapeDtypeStruct((NP, IN_C), jnp.float32),
        ],
    )(degp, x_p)


def _mid_body(p_ref, g0_ref, dinv_ref, W1_ref, b1_ref, W2_ref, g1_ref):
    dinv = jnp.broadcast_to(dinv_ref[...], (_RB, IN_C))
    P0 = dinv * (p_ref[0] + p_ref[1] + g0_ref[...])
    h1 = jnp.maximum(
        jnp.dot(P0.astype(jnp.bfloat16), W1_ref[...].astype(jnp.bfloat16),
                preferred_element_type=jnp.float32)
        + b1_ref[...], 0.0)
    q = jnp.dot(h1.astype(jnp.bfloat16), W2_ref[...].astype(jnp.bfloat16),
                preferred_element_type=jnp.float32)
    g1_ref[...] = q * dinv


def _mid_call(p, g0, dinv, W1, b1, W2):
    grid = NP // _RB
    return pl.pallas_call(
        _mid_body,
        grid=(grid,),
        in_specs=[
            pl.BlockSpec((2, _RB, IN_C), lambda i: (0, i, 0)),
            pl.BlockSpec((_RB, IN_C), lambda i: (i, 0)),
            pl.BlockSpec((_RB, 1), lambda i: (i, 0)),
            pl.BlockSpec((IN_C, HID_C), lambda i: (0, 0)),
            pl.BlockSpec((1, HID_C), lambda i: (0, 0)),
            pl.BlockSpec((HID_C, OUT_C), lambda i: (0, 0)),
        ],
        out_specs=pl.BlockSpec((_RB, OUT_C), lambda i: (i, 0)),
        out_shape=jax.ShapeDtypeStruct((NP, OUT_C), jnp.float32),
    )(p, g0, dinv, W1, b1, W2)


_RBF = 2000


def _fin_body(q_ref, g1_ref, dinv_ref, b2_ref, z_ref):
    dinv = jnp.broadcast_to(dinv_ref[...], (_RBF, OUT_C))
    z_ref[...] = dinv * (q_ref[0] + q_ref[1] + g1_ref[...]) + b2_ref[...]


def _fin_call(q, g1, dinv, b2):
    grid = N // _RBF
    return pl.pallas_call(
        _fin_body,
        grid=(grid,),
        in_specs=[
            pl.BlockSpec((2, _RBF, OUT_C), lambda i: (0, i, 0)),
            pl.BlockSpec((_RBF, OUT_C), lambda i: (i, 0)),
            pl.BlockSpec((_RBF, 1), lambda i: (i, 0)),
            pl.BlockSpec((1, OUT_C), lambda i: (0, 0)),
        ],
        out_specs=pl.BlockSpec((_RBF, OUT_C), lambda i: (i, 0)),
        out_shape=jax.ShapeDtypeStruct((N, OUT_C), jnp.float32),
    )(q, g1, dinv, b2)


def kernel(x, edge_index, W1, b1, W2, b2):
    src = edge_index[0]
    dst = edge_index[1]
    npad = EPAD - E
    spread = jnp.arange(npad, dtype=jnp.int32) % 128
    src_p = jnp.concatenate([src, spread])
    dst_p = jnp.concatenate([dst, N + spread])

    z128 = jnp.zeros((RPT, IN_C), jnp.float32)

    src2d = src_p.reshape(NW * NCHUNK, CH)
    dst2d = dst_p.reshape(NW * NCHUNK, CH)
    degp = _deg_call(edge_index.reshape(2 * E)).reshape(2, NP // 128, 128)
    dinv, g0 = _prep_call(degp, x)
    p0 = _agg_call(src2d, dst2d, g0, z128).reshape(2, NP, IN_C)
    g1 = _mid_call(p0, g0, dinv, W1, b1.reshape(1, HID_C), W2)
    p1 = _agg_call(src2d, dst2d, g1, z128).reshape(2, NP, OUT_C)
    z = _fin_call(p1, g1, dinv, b2.reshape(1, OUT_C))
    return z


# speedup vs baseline: 1.0017x; 1.0017x over previous
import functools

import jax
import jax.numpy as jnp
from jax import lax
from jax.experimental import pallas as pl
from jax.experimental.pallas import tpu as pltpu
from jax.experimental.pallas import tpu_sc as plsc

N = 10000
NP = 10240
E = 320000
IN_C = 128
HID_C = 256
OUT_C = 128

CH = 128
NC = 2
NS = 16
NW = NC * NS
NCHUNK = 80
HC = 40
EPW = NCHUNK * CH
EPAD = EPW * NW
RPT = NP // NS

_sc_mesh = plsc.VectorSubcoreMesh(core_axis_name="c", subcore_axis_name="s")


EDGW = E // NW


def _deg_body(ei_hbm, out_hbm, didx_all, deg_local, colbuf, shared):
    cid = lax.axis_index("c")
    sid = lax.axis_index("s")
    wid = sid * NC + cid
    zeros = jnp.zeros((16,), jnp.float32)

    def zbody(i, c):
        deg_local[pl.ds(i * 16, 16)] = zeros
        return c

    lax.fori_loop(0, NP // 16, zbody, 0)
    pltpu.sync_copy(ei_hbm.at[pl.ds(E + wid * EDGW, EDGW)], didx_all)
    ones = jnp.ones((16,), jnp.float32)

    def body(i, c):
        idx = didx_all[pl.ds(i * 16, 16)]
        plsc.addupdate_scatter(deg_local, [idx], ones)
        return c

    lax.fori_loop(0, EDGW // 16, body, 0)
    pltpu.sync_copy(deg_local, shared.at[sid])
    plsc.subcore_barrier()
    for t in range(NS):
        pltpu.sync_copy(shared.at[t, pl.ds(sid * RPT, RPT)], colbuf.at[t])

    def red(j, c):
        s = colbuf[0, pl.ds(j * 16, 16)]
        for t in range(1, NS):
            s = s + colbuf[t, pl.ds(j * 16, 16)]
        deg_local[pl.ds(j * 16, 16)] = s
        return c

    lax.fori_loop(0, RPT // 16, red, 0)
    pltpu.sync_copy(deg_local.at[pl.ds(0, RPT)],
                    out_hbm.at[pl.ds(cid * NP + sid * RPT, RPT)])


_deg_call = pl.kernel(
    _deg_body,
    out_type=jax.ShapeDtypeStruct((2 * NP,), jnp.float32),
    mesh=_sc_mesh,
    compiler_params=pltpu.CompilerParams(needs_layout_passes=False),
    scratch_types=[
        pltpu.VMEM((EDGW,), jnp.int32),
        pltpu.VMEM((NP,), jnp.float32),
        pltpu.VMEM((NS, RPT), jnp.float32),
        pltpu.VMEM_SHARED((NS, NP), jnp.float32),
    ],
)


def _agg_body(src_hbm, dst_hbm, g_hbm, zrows_hbm, out_hbm,
              sidx, didx, rows0, rows1, acc, sem0, sem1):
    cid = lax.axis_index("c")
    sid = lax.axis_index("s")
    wid = sid * NC + cid
    pltpu.sync_copy(zrows_hbm, acc.at[pl.ds(sid * RPT, RPT)])
    plsc.subcore_barrier()
    hpair = HC // 2
    for sb in range(NCHUNK // HC):
        base_c = wid * NCHUNK + sb * HC
        pltpu.sync_copy(src_hbm.at[pl.ds(base_c, HC)], sidx)
        pltpu.sync_copy(dst_hbm.at[pl.ds(base_c, HC)], didx)
        pltpu.async_copy(g_hbm.at[sidx.at[0]], rows0, sem0)

        def body(j, carry):
            c = 4 * j
            pltpu.async_copy(g_hbm.at[sidx.at[c + 1]], rows1, sem1)
            pltpu.make_async_copy(g_hbm.at[sidx.at[c]], rows0, sem0).wait()
            pltpu.sync_copy(rows0, acc.at[didx.at[c]], add=True)
            pltpu.async_copy(g_hbm.at[sidx.at[c + 2]], rows0, sem0)
            pltpu.make_async_copy(g_hbm.at[sidx.at[c + 1]], rows1, sem1).wait()
            pltpu.sync_copy(rows1, acc.at[didx.at[c + 1]], add=True)
            pltpu.async_copy(g_hbm.at[sidx.at[c + 3]], rows1, sem1)
            pltpu.make_async_copy(g_hbm.at[sidx.at[c + 2]], rows0, sem0).wait()
            pltpu.sync_copy(rows0, acc.at[didx.at[c + 2]], add=True)

            @pl.when(j < hpair // 2 - 1)
            def _():
                pltpu.async_copy(g_hbm.at[sidx.at[c + 4]], rows0, sem0)

            pltpu.make_async_copy(g_hbm.at[sidx.at[c + 3]], rows1, sem1).wait()
            pltpu.sync_copy(rows1, acc.at[didx.at[c + 3]], add=True)
            return carry

        lax.fori_loop(0, hpair // 2, body, 0)
    plsc.subcore_barrier()
    pltpu.sync_copy(acc.at[pl.ds(sid * RPT, RPT)],
                    out_hbm.at[pl.ds(cid * NP + sid * RPT, RPT)])


_agg_call = pl.kernel(
    _agg_body,
    out_type=jax.ShapeDtypeStruct((2 * NP, IN_C), jnp.float32),
    mesh=_sc_mesh,
    scratch_types=[
        pltpu.VMEM((HC, CH), jnp.int32),
        pltpu.VMEM((HC, CH), jnp.int32),
        pltpu.VMEM((CH, IN_C), jnp.float32),
        pltpu.VMEM((CH, IN_C), jnp.float32),
        pltpu.VMEM_SHARED((NP, IN_C), jnp.float32),
        pltpu.SemaphoreType.DMA,
        pltpu.SemaphoreType.DMA,
    ],
)


_RB = 2048


def _prep_body(degp_ref, x_ref, dinv_ref, g0_ref):
    nrow = _RB // 128
    deg = degp_ref[0] + degp_ref[1] + 1.0
    dinv = lax.rsqrt(jnp.maximum(deg, 1.0))
    drep = jnp.broadcast_to(dinv[:, None, :], (nrow, 128, 128)).reshape(_RB, 128)
    lane = lax.broadcasted_iota(jnp.int32, (_RB, 128), 1)
    row = lax.broadcasted_iota(jnp.int32, (_RB, 128), 0)
    sel = jnp.where(lane == row % 128, drep, 0.0)
    dinv_col = jnp.sum(sel, axis=1, keepdims=True)
    dinv_ref[...] = dinv_col
    g0_ref[...] = x_ref[...] * jnp.broadcast_to(dinv_col, (_RB, IN_C))


def _prep_call(degp, x_p):
    grid = NP // _RB
    return pl.pallas_call(
        _prep_body,
        grid=(grid,),
        in_specs=[
            pl.BlockSpec((2, _RB // 128, 128), lambda i: (0, i, 0)),
            pl.BlockSpec((_RB, IN_C), lambda i: (i, 0)),
        ],
        out_specs=[
            pl.BlockSpec((_RB, 1), lambda i: (i, 0)),
            pl.BlockSpec((_RB, IN_C), lambda i: (i, 0)),
        ],
        out_shape=[
            jax.ShapeDtypeStruct((NP, 1), jnp.float32),
            jax.ShapeDtypeStruct((NP, IN_C), jnp.float32),
        ],
    )(degp, x_p)


def _mid_body(p_ref, g0_ref, dinv_ref, W1_ref, b1_ref, W2_ref, g1_ref):
    dinv = jnp.broadcast_to(dinv_ref[...], (_RB, IN_C))
    P0 = dinv * (p_ref[0] + p_ref[1] + g0_ref[...])
    h1 = jnp.maximum(
        jnp.dot(P0, W1_ref[...], preferred_element_type=jnp.float32)
        + b1_ref[...], 0.0)
    q = jnp.dot(h1, W2_ref[...], preferred_element_type=jnp.float32)
    g1_ref[...] = q * dinv


def _mid_call(p, g0, dinv, W1, b1, W2):
    grid = NP // _RB
    return pl.pallas_call(
        _mid_body,
        grid=(grid,),
        in_specs=[
            pl.BlockSpec((2, _RB, IN_C), lambda i: (0, i, 0)),
            pl.BlockSpec((_RB, IN_C), lambda i: (i, 0)),
            pl.BlockSpec((_RB, 1), lambda i: (i, 0)),
            pl.BlockSpec((IN_C, HID_C), lambda i: (0, 0)),
            pl.BlockSpec((1, HID_C), lambda i: (0, 0)),
            pl.BlockSpec((HID_C, OUT_C), lambda i: (0, 0)),
        ],
        out_specs=pl.BlockSpec((_RB, OUT_C), lambda i: (i, 0)),
        out_shape=jax.ShapeDtypeStruct((NP, OUT_C), jnp.float32),
    )(p, g0, dinv, W1, b1, W2)


_RBF = 2000


def _fin_body(q_ref, g1_ref, dinv_ref, b2_ref, z_ref):
    dinv = jnp.broadcast_to(dinv_ref[...], (_RBF, OUT_C))
    z_ref[...] = dinv * (q_ref[0] + q_ref[1] + g1_ref[...]) + b2_ref[...]


def _fin_call(q, g1, dinv, b2):
    grid = N // _RBF
    return pl.pallas_call(
        _fin_body,
        grid=(grid,),
        in_specs=[
            pl.BlockSpec((2, _RBF, OUT_C), lambda i: (0, i, 0)),
            pl.BlockSpec((_RBF, OUT_C), lambda i: (i, 0)),
            pl.BlockSpec((_RBF, 1), lambda i: (i, 0)),
            pl.BlockSpec((1, OUT_C), lambda i: (0, 0)),
        ],
        out_specs=pl.BlockSpec((_RBF, OUT_C), lambda i: (i, 0)),
        out_shape=jax.ShapeDtypeStruct((N, OUT_C), jnp.float32),
    )(q, g1, dinv, b2)


def kernel(x, edge_index, W1, b1, W2, b2):
    src = edge_index[0]
    dst = edge_index[1]
    npad = EPAD - E
    spread = jnp.arange(npad, dtype=jnp.int32) % 128
    src_p = jnp.concatenate([src, spread])
    dst_p = jnp.concatenate([dst, N + spread])

    z128 = jnp.zeros((RPT, IN_C), jnp.float32)

    src2d = src_p.reshape(NW * NCHUNK, CH)
    dst2d = dst_p.reshape(NW * NCHUNK, CH)
    degp = _deg_call(edge_index.reshape(2 * E)).reshape(2, NP // 128, 128)
    dinv, g0 = _prep_call(degp, x)
    p0 = _agg_call(src2d, dst2d, g0, z128).reshape(2, NP, IN_C)
    g1 = _mid_call(p0, g0, dinv, W1, b1.reshape(1, HID_C), W2)
    p1 = _agg_call(src2d, dst2d, g1, z128).reshape(2, NP, OUT_C)
    z = _fin_call(p1, g1, dinv, b2.reshape(1, OUT_C))
    return z
